# Initial kernel scaffold; baseline (speedup 1.0000x reference)
import jax
import jax.numpy as jnp
from jax.experimental import pallas as pl


def kernel(x, edge_index, conv1_w, conv1_b, conv2_w, conv2_b, gcn1_W, gcn1_b, gcn2_W, gcn2_b, cls_W, cls_b):
    raise NotImplementedError("write your pallas kernel here")


# speedup vs baseline: 13.1344x; 13.1344x over previous
import jax
import jax.numpy as jnp
from jax import lax
from jax.experimental import pallas as pl
from jax.experimental.pallas import tpu as pltpu
from jax.experimental.pallas import tpu_sc as plsc

B = 16
N_NODES = 32768
N_EDGES = 524288
NC = 2
NS = 16
CHUNK = 128
ROWS_PER_TILE = N_NODES // NS

PREC = lax.Precision.HIGHEST


def _conv1_body(xs_ref, m_ref, b_ref, out_ref):
    a = xs_ref[0]
    r = (jnp.dot(a[1:4097], m_ref[0], precision=PREC)
         + jnp.dot(a[2:4098], m_ref[1], precision=PREC)
         + jnp.dot(a[3:4099], m_ref[2], precision=PREC))
    out_ref[0] = jnp.maximum(
        jnp.maximum(r[:, :16], r[:, 16:]) + b_ref[...], 0.0)


def _conv2_body(h_ref, n_ref, b_ref, w_ref, out_ref):
    a = h_ref[0]
    r = (jnp.dot(a[1:2049], n_ref[0], precision=PREC)
         + jnp.dot(a[2:2050], n_ref[1], precision=PREC)
         + jnp.dot(a[3:2051], n_ref[2], precision=PREC))
    p = jnp.maximum(jnp.maximum(r[:, :32], r[:, 32:]) + b_ref[...], 0.0)
    xw = jnp.dot(p, w_ref[...], precision=PREC)
    out_ref[0] = xw[:, :32]
    out_ref[1] = xw[:, 32:]


def _scale_body(degp_ref, xw_ref, dis_ref, y_ref):
    d = degp_ref[0, :, 0:1] + degp_ref[1, :, 0:1] + 1.0
    dis = lax.rsqrt(d)
    dis_ref[...] = dis
    y_ref[0] = dis * xw_ref[0]
    y_ref[1] = dis * xw_ref[1]


def _layer_body(s_ref, dis_ref, b_ref, w_ref, y_ref):
    dis = dis_ref[...]
    g = jnp.maximum(
        dis * jnp.concatenate([s_ref[0], s_ref[1]], axis=1) + b_ref[...], 0.0)
    y = dis * jnp.dot(g, w_ref[...], precision=PREC)
    y_ref[0] = y[:, :32]
    y_ref[1] = y[:, 32:]


def _head_body(s_ref, dis_ref, b_ref, w_ref, cb_ref, out_ref):
    dis = dis_ref[...]
    g = jnp.maximum(
        dis * jnp.concatenate([s_ref[0], s_ref[1]], axis=1) + b_ref[...], 0.0)
    m = jnp.mean(g, axis=0, keepdims=True)
    out_ref[0] = jnp.dot(m, w_ref[...], precision=PREC) + cb_ref[...]


def _deg_kernel(edge_hbm, degp_hbm, spmem, idx_v, ones_v, zero_v):
    c = lax.axis_index("c")
    s = lax.axis_index("s")

    @pl.loop(0, CHUNK)
    def _(i):
        ones_v[i] = jnp.ones((16,), jnp.float32)
        zero_v[i] = jnp.zeros((16,), jnp.float32)

    @pl.loop(0, ROWS_PER_TILE // CHUNK)
    def _(t):
        pltpu.sync_copy(zero_v,
                        spmem.at[pl.ds(s * ROWS_PER_TILE + t * CHUNK, CHUNK)])

    plsc.subcore_barrier()

    per_worker = N_EDGES // (NC * NS)
    base = (c * NS + s) * per_worker

    @pl.loop(0, per_worker // CHUNK)
    def _(i):
        pltpu.sync_copy(edge_hbm.at[1].at[pl.ds(base + i * CHUNK, CHUNK)],
                        idx_v)
        pltpu.sync_copy(ones_v, spmem.at[idx_v], add=True)

    plsc.subcore_barrier()
    pltpu.sync_copy(spmem.at[pl.ds(s * ROWS_PER_TILE, ROWS_PER_TILE)],
                    degp_hbm.at[c].at[pl.ds(s * ROWS_PER_TILE, ROWS_PER_TILE)])


def _agg_kernel(y_hbm, edge_hbm, s_hbm, spmem, src_v, adj_v, dst_v, val_v,
                sem):
    c = lax.axis_index("c")
    s = lax.axis_index("s")

    pltpu.sync_copy(
        y_hbm.at[pl.ds(c * N_NODES + s * ROWS_PER_TILE, ROWS_PER_TILE)],
        spmem.at[pl.ds(s * ROWS_PER_TILE, ROWS_PER_TILE)])
    plsc.subcore_barrier()

    off = c * N_NODES
    per_tile = N_EDGES // NS
    base = s * per_tile

    @pl.loop(0, per_tile // CHUNK)
    def _(i):
        e0 = base + i * CHUNK
        pltpu.sync_copy(edge_hbm.at[0].at[pl.ds(e0, CHUNK)], src_v)
        pltpu.sync_copy(edge_hbm.at[1].at[pl.ds(e0, CHUNK)], dst_v)

        @pl.loop(0, CHUNK // 16)
        def _(j):
            adj_v[pl.ds(j * 16, 16)] = src_v[pl.ds(j * 16, 16)] + off

        pltpu.async_copy(y_hbm.at[adj_v], val_v, sem).wait()
        pltpu.sync_copy(val_v, spmem.at[dst_v], add=True)

    plsc.subcore_barrier()
    pltpu.sync_copy(spmem.at[pl.ds(s * ROWS_PER_TILE, ROWS_PER_TILE)],
                    s_hbm.at[c].at[pl.ds(s * ROWS_PER_TILE, ROWS_PER_TILE)])


def _sc_mesh():
    return plsc.VectorSubcoreMesh(core_axis_name="c", subcore_axis_name="s")


_SC_PARAMS = pltpu.CompilerParams(use_tc_tiling_on_sc=False)


def _sc_degree(edge_index):
    return pl.kernel(
        _deg_kernel,
        out_type=jax.ShapeDtypeStruct((NC, N_NODES, 16), jnp.float32),
        mesh=_sc_mesh(),
        scratch_types=[
            pltpu.VMEM_SHARED((N_NODES, 16), jnp.float32),
            pltpu.VMEM((CHUNK,), jnp.int32),
            pltpu.VMEM((CHUNK, 16), jnp.float32),
            pltpu.VMEM((CHUNK, 16), jnp.float32),
        ],
        compiler_params=_SC_PARAMS,
    )(edge_index)


def _sc_aggregate(y_flat, edge_index):
    return pl.kernel(
        _agg_kernel,
        out_type=jax.ShapeDtypeStruct((NC, N_NODES, 32), jnp.float32),
        mesh=_sc_mesh(),
        scratch_types=[
            pltpu.VMEM_SHARED((N_NODES, 32), jnp.float32),
            pltpu.VMEM((CHUNK,), jnp.int32),
            pltpu.VMEM((CHUNK,), jnp.int32),
            pltpu.VMEM((CHUNK,), jnp.int32),
            pltpu.VMEM((CHUNK, 32), jnp.float32),
            pltpu.SemaphoreType.DMA,
        ],
        compiler_params=_SC_PARAMS,
    )(y_flat, edge_index)


def _stack_taps(wt, z):
    m1 = jnp.concatenate([jnp.concatenate([wt[0], z], 1),
                          jnp.concatenate([wt[1], wt[0]], 1)], 0)
    m2 = jnp.concatenate([jnp.concatenate([wt[2], wt[1]], 1),
                          jnp.concatenate([wt[3], wt[2]], 1)], 0)
    m3 = jnp.concatenate([jnp.concatenate([wt[4], wt[3]], 1),
                          jnp.concatenate([z, wt[4]], 1)], 0)
    return jnp.stack([m1, m2, m3])


def kernel(x, edge_index, conv1_w, conv1_b, conv2_w, conv2_b,
           gcn1_W, gcn1_b, gcn2_W, gcn2_b, cls_W, cls_b):
    f32 = jnp.float32

    wt1 = jnp.transpose(conv1_w, (2, 1, 0))
    m = _stack_taps(wt1, jnp.zeros_like(wt1[0]))
    wt2 = jnp.transpose(conv2_w, (2, 1, 0))
    n = _stack_taps(wt2, jnp.zeros_like(wt2[0]))

    xs = jnp.pad(jnp.transpose(x, (0, 2, 1)),
                 ((0, 0), (4, 12), (0, 0))).reshape(B, 4104, 128)

    degp = _sc_degree(edge_index)

    h1p = pl.pallas_call(
        _conv1_body,
        grid=(B,),
        in_specs=[pl.BlockSpec((1, 4104, 128), lambda b: (b, 0, 0)),
                  pl.BlockSpec((3, 128, 32), lambda b: (0, 0, 0)),
                  pl.BlockSpec((1, 16), lambda b: (0, 0))],
        out_specs=pl.BlockSpec((1, 4096, 16), lambda b: (b, 0, 0)),
        out_shape=jax.ShapeDtypeStruct((B, 4096, 16), f32),
    )(xs, m, conv1_b.reshape(1, 16))

    h1s = jnp.pad(h1p, ((0, 0), (4, 12), (0, 0))).reshape(B, 2056, 32)

    xw1 = pl.pallas_call(
        _conv2_body,
        grid=(B,),
        in_specs=[pl.BlockSpec((1, 2056, 32), lambda b: (b, 0, 0)),
                  pl.BlockSpec((3, 32, 64), lambda b: (0, 0, 0)),
                  pl.BlockSpec((1, 32), lambda b: (0, 0)),
                  pl.BlockSpec((32, 64), lambda b: (0, 0))],
        out_specs=pl.BlockSpec((2, 2048, 32), lambda b: (0, b, 0)),
        out_shape=jax.ShapeDtypeStruct((NC, N_NODES, 32), f32),
    )(h1s, n, conv2_b.reshape(1, 32), gcn1_W)

    dis, y1 = pl.pallas_call(
        _scale_body,
        grid=(16,),
        in_specs=[pl.BlockSpec((2, 2048, 16), lambda i: (0, i, 0)),
                  pl.BlockSpec((2, 2048, 32), lambda i: (0, i, 0))],
        out_specs=[pl.BlockSpec((2048, 1), lambda i: (i, 0)),
                   pl.BlockSpec((2, 2048, 32), lambda i: (0, i, 0))],
        out_shape=[jax.ShapeDtypeStruct((N_NODES, 1), f32),
                   jax.ShapeDtypeStruct((NC, N_NODES, 32), f32)],
    )(degp, xw1)

    s1 = _sc_aggregate(y1.reshape(NC * N_NODES, 32), edge_index)

    y2 = pl.pallas_call(
        _layer_body,
        grid=(16,),
        in_specs=[pl.BlockSpec((2, 2048, 32), lambda i: (0, i, 0)),
                  pl.BlockSpec((2048, 1), lambda i: (i, 0)),
                  pl.BlockSpec((1, 64), lambda i: (0, 0)),
                  pl.BlockSpec((64, 64), lambda i: (0, 0))],
        out_specs=pl.BlockSpec((2, 2048, 32), lambda i: (0, i, 0)),
        out_shape=jax.ShapeDtypeStruct((NC, N_NODES, 32), f32),
    )(s1, dis, gcn1_b.reshape(1, 64), gcn2_W)

    s2 = _sc_aggregate(y2.reshape(NC * N_NODES, 32), edge_index)

    cls_wp = jnp.pad(cls_W, ((0, 0), (0, 118)))
    cls_bp = jnp.pad(cls_b, (0, 118)).reshape(1, 128)
    outp = pl.pallas_call(
        _head_body,
        grid=(B,),
        in_specs=[pl.BlockSpec((2, 2048, 32), lambda b: (0, b, 0)),
                  pl.BlockSpec((2048, 1), lambda b: (b, 0)),
                  pl.BlockSpec((1, 64), lambda b: (0, 0)),
                  pl.BlockSpec((64, 128), lambda b: (0, 0)),
                  pl.BlockSpec((1, 128), lambda b: (0, 0))],
        out_specs=pl.BlockSpec((1, 1, 128), lambda b: (b, 0, 0)),
        out_shape=jax.ShapeDtypeStruct((B, 1, 128), f32),
    )(s2, dis, gcn2_b.reshape(1, 64), cls_wp, cls_bp)

    return outp[:, 0, :10]


# speedup vs baseline: 24.2109x; 1.8433x over previous
import jax
import jax.numpy as jnp
from jax import lax
from jax.experimental import pallas as pl
from jax.experimental.pallas import tpu as pltpu
from jax.experimental.pallas import tpu_sc as plsc

B = 16
N_NODES = 32768
N_EDGES = 524288
NC = 2
NS = 16
CHUNK = 128
ROWS_PER_TILE = N_NODES // NS

PREC = lax.Precision.HIGHEST


def _conv1_body(xs_ref, m_ref, b_ref, out_ref):
    a = xs_ref[0]
    r = (jnp.dot(a[1:4097], m_ref[0], precision=PREC)
         + jnp.dot(a[2:4098], m_ref[1], precision=PREC)
         + jnp.dot(a[3:4099], m_ref[2], precision=PREC))
    out_ref[0] = jnp.maximum(
        jnp.maximum(r[:, :16], r[:, 16:]) + b_ref[...], 0.0)


def _conv2_body(h_ref, n_ref, b_ref, w_ref, out_ref):
    a = h_ref[0]
    r = (jnp.dot(a[1:2049], n_ref[0], precision=PREC)
         + jnp.dot(a[2:2050], n_ref[1], precision=PREC)
         + jnp.dot(a[3:2051], n_ref[2], precision=PREC))
    p = jnp.maximum(jnp.maximum(r[:, :32], r[:, 32:]) + b_ref[...], 0.0)
    xw = jnp.dot(p, w_ref[...], precision=PREC)
    out_ref[0] = xw[:, :32]
    out_ref[1] = xw[:, 32:]


def _scale_body(degp_ref, xw_ref, src_ref, dis_ref, y_ref, srcs_ref):
    d = degp_ref[0, :, 0:1] + degp_ref[1, :, 0:1] + 1.0
    dis = lax.rsqrt(d)
    dis_ref[...] = dis
    y_ref[0] = dis * xw_ref[0]
    y_ref[1] = dis * xw_ref[1]
    e = src_ref[...]
    srcs_ref[0] = e
    srcs_ref[1] = e + N_NODES


def _layer_body(s_ref, dis_ref, b_ref, w_ref, y_ref):
    dis = dis_ref[...]
    g = jnp.maximum(
        dis * jnp.concatenate([s_ref[0], s_ref[1]], axis=1) + b_ref[...], 0.0)
    y = dis * jnp.dot(g, w_ref[...], precision=PREC)
    y_ref[0] = y[:, :32]
    y_ref[1] = y[:, 32:]


def _head_body(s_ref, dis_ref, b_ref, w_ref, cb_ref, out_ref):
    dis = dis_ref[...]
    g = jnp.maximum(
        dis * jnp.concatenate([s_ref[0], s_ref[1]], axis=1) + b_ref[...], 0.0)
    m = jnp.mean(g, axis=0, keepdims=True)
    out_ref[0] = jnp.dot(m, w_ref[...], precision=PREC) + cb_ref[...]


def _deg_kernel(dst_hbm, degp_hbm, spmem, dstd2, ones_v, zero_v):
    c = lax.axis_index("c")
    s = lax.axis_index("s")
    w = c * NS + s

    @pl.loop(0, CHUNK)
    def _(i):
        ones_v[i] = jnp.ones((16,), jnp.float32)
        zero_v[i] = jnp.zeros((16,), jnp.float32)

    pltpu.sync_copy(dst_hbm.at[pl.ds(w * 128, 128)], dstd2)

    @pl.loop(0, ROWS_PER_TILE // CHUNK)
    def _(t):
        pltpu.sync_copy(zero_v,
                        spmem.at[pl.ds(s * ROWS_PER_TILE + t * CHUNK, CHUNK)])

    plsc.subcore_barrier()

    @pl.loop(0, 128)
    def _(i):
        pltpu.sync_copy(ones_v, spmem.at[dstd2.at[i]], add=True)

    plsc.subcore_barrier()
    pltpu.sync_copy(spmem.at[pl.ds(s * ROWS_PER_TILE, ROWS_PER_TILE)],
                    degp_hbm.at[c].at[pl.ds(s * ROWS_PER_TILE, ROWS_PER_TILE)])


def _agg_kernel(y_hbm, srcs_hbm, dst_hbm, s_hbm, spmem,
                adj_a, dst_a, vals_a, adj_b, dst_b, vals_b, sem_a, sem_b):
    c = lax.axis_index("c")
    s = lax.axis_index("s")
    row0 = s * ROWS_PER_TILE
    erow0 = s * 256

    init = pltpu.async_copy(y_hbm.at[pl.ds(c * N_NODES + row0, ROWS_PER_TILE)],
                            spmem.at[pl.ds(row0, ROWS_PER_TILE)], sem_a)

    def load_idx(g, adj, dst):
        pltpu.sync_copy(srcs_hbm.at[c].at[pl.ds(erow0 + g * 4, 4)], adj)
        pltpu.sync_copy(dst_hbm.at[pl.ds(erow0 + g * 4, 4)], dst)

    def fire(adj, vals, sem):
        for b in range(4):
            pltpu.async_copy(y_hbm.at[adj.at[b]], vals.at[b], sem)

    def drain(adj, vals, sem):
        for b in range(4):
            pltpu.make_async_copy(y_hbm.at[adj.at[b]], vals.at[b],
                                  sem).wait()

    def scat(dst, vals):
        for b in range(4):
            pltpu.sync_copy(vals.at[b], spmem.at[dst.at[b]], add=True)

    load_idx(0, adj_a, dst_a)
    init.wait()
    plsc.subcore_barrier()
    fire(adj_a, vals_a, sem_a)

    @pl.loop(0, 64, step=2)
    def _(g):
        load_idx(g + 1, adj_b, dst_b)
        fire(adj_b, vals_b, sem_b)
        drain(adj_a, vals_a, sem_a)
        scat(dst_a, vals_a)

        @pl.when(g + 2 < 64)
        def _():
            load_idx(g + 2, adj_a, dst_a)
            fire(adj_a, vals_a, sem_a)

        drain(adj_b, vals_b, sem_b)
        scat(dst_b, vals_b)

    plsc.subcore_barrier()
    pltpu.sync_copy(spmem.at[pl.ds(row0, ROWS_PER_TILE)],
                    s_hbm.at[c].at[pl.ds(row0, ROWS_PER_TILE)])


def _sc_mesh():
    return plsc.VectorSubcoreMesh(core_axis_name="c", subcore_axis_name="s")


_SC_PARAMS = pltpu.CompilerParams(use_tc_tiling_on_sc=False)


def _sc_degree(dst_r):
    return pl.kernel(
        _deg_kernel,
        out_type=jax.ShapeDtypeStruct((NC, N_NODES, 16), jnp.float32),
        mesh=_sc_mesh(),
        scratch_types=[
            pltpu.VMEM_SHARED((N_NODES, 16), jnp.float32),
            pltpu.VMEM((128, CHUNK), jnp.int32),
            pltpu.VMEM((CHUNK, 16), jnp.float32),
            pltpu.VMEM((CHUNK, 16), jnp.float32),
        ],
        compiler_params=_SC_PARAMS,
    )(dst_r)


def _sc_aggregate(y_flat, srcs, dst_r):
    return pl.kernel(
        _agg_kernel,
        out_type=jax.ShapeDtypeStruct((NC, N_NODES, 32), jnp.float32),
        mesh=_sc_mesh(),
        scratch_types=[
            pltpu.VMEM_SHARED((N_NODES, 32), jnp.float32),
            pltpu.VMEM((4, CHUNK), jnp.int32),
            pltpu.VMEM((4, CHUNK), jnp.int32),
            pltpu.VMEM((4, CHUNK, 32), jnp.float32),
            pltpu.VMEM((4, CHUNK), jnp.int32),
            pltpu.VMEM((4, CHUNK), jnp.int32),
            pltpu.VMEM((4, CHUNK, 32), jnp.float32),
            pltpu.SemaphoreType.DMA,
            pltpu.SemaphoreType.DMA,
        ],
        compiler_params=_SC_PARAMS,
    )(y_flat, srcs, dst_r)


def _stack_taps(wt, z):
    m1 = jnp.concatenate([jnp.concatenate([wt[0], z], 1),
                          jnp.concatenate([wt[1], wt[0]], 1)], 0)
    m2 = jnp.concatenate([jnp.concatenate([wt[2], wt[1]], 1),
                          jnp.concatenate([wt[3], wt[2]], 1)], 0)
    m3 = jnp.concatenate([jnp.concatenate([wt[4], wt[3]], 1),
                          jnp.concatenate([z, wt[4]], 1)], 0)
    return jnp.stack([m1, m2, m3])


def kernel(x, edge_index, conv1_w, conv1_b, conv2_w, conv2_b,
           gcn1_W, gcn1_b, gcn2_W, gcn2_b, cls_W, cls_b):
    f32 = jnp.float32

    wt1 = jnp.transpose(conv1_w, (2, 1, 0))
    m = _stack_taps(wt1, jnp.zeros_like(wt1[0]))
    wt2 = jnp.transpose(conv2_w, (2, 1, 0))
    n = _stack_taps(wt2, jnp.zeros_like(wt2[0]))

    xs = jnp.pad(jnp.transpose(x, (0, 2, 1)),
                 ((0, 0), (4, 12), (0, 0))).reshape(B, 4104, 128)
    src_r = edge_index[0].reshape(4096, 128)
    dst_r = edge_index[1].reshape(4096, 128)

    degp = _sc_degree(dst_r)

    h1p = pl.pallas_call(
        _conv1_body,
        grid=(B,),
        in_specs=[pl.BlockSpec((1, 4104, 128), lambda b: (b, 0, 0)),
                  pl.BlockSpec((3, 128, 32), lambda b: (0, 0, 0)),
                  pl.BlockSpec((1, 16), lambda b: (0, 0))],
        out_specs=pl.BlockSpec((1, 4096, 16), lambda b: (b, 0, 0)),
        out_shape=jax.ShapeDtypeStruct((B, 4096, 16), f32),
    )(xs, m, conv1_b.reshape(1, 16))

    h1s = jnp.pad(h1p, ((0, 0), (4, 12), (0, 0))).reshape(B, 2056, 32)

    xw1 = pl.pallas_call(
        _conv2_body,
        grid=(B,),
        in_specs=[pl.BlockSpec((1, 2056, 32), lambda b: (b, 0, 0)),
                  pl.BlockSpec((3, 32, 64), lambda b: (0, 0, 0)),
                  pl.BlockSpec((1, 32), lambda b: (0, 0)),
                  pl.BlockSpec((32, 64), lambda b: (0, 0))],
        out_specs=pl.BlockSpec((2, 2048, 32), lambda b: (0, b, 0)),
        out_shape=jax.ShapeDtypeStruct((NC, N_NODES, 32), f32),
    )(h1s, n, conv2_b.reshape(1, 32), gcn1_W)

    dis, y1, srcs = pl.pallas_call(
        _scale_body,
        grid=(16,),
        in_specs=[pl.BlockSpec((2, 2048, 16), lambda i: (0, i, 0)),
                  pl.BlockSpec((2, 2048, 32), lambda i: (0, i, 0)),
                  pl.BlockSpec((256, 128), lambda i: (i, 0))],
        out_specs=[pl.BlockSpec((2048, 1), lambda i: (i, 0)),
                   pl.BlockSpec((2, 2048, 32), lambda i: (0, i, 0)),
                   pl.BlockSpec((2, 256, 128), lambda i: (0, i, 0))],
        out_shape=[jax.ShapeDtypeStruct((N_NODES, 1), f32),
                   jax.ShapeDtypeStruct((NC, N_NODES, 32), f32),
                   jax.ShapeDtypeStruct((NC, 4096, 128), jnp.int32)],
    )(degp, xw1, src_r)

    s1 = _sc_aggregate(y1.reshape(NC * N_NODES, 32), srcs, dst_r)

    y2 = pl.pallas_call(
        _layer_body,
        grid=(16,),
        in_specs=[pl.BlockSpec((2, 2048, 32), lambda i: (0, i, 0)),
                  pl.BlockSpec((2048, 1), lambda i: (i, 0)),
                  pl.BlockSpec((1, 64), lambda i: (0, 0)),
                  pl.BlockSpec((64, 64), lambda i: (0, 0))],
        out_specs=pl.BlockSpec((2, 2048, 32), lambda i: (0, i, 0)),
        out_shape=jax.ShapeDtypeStruct((NC, N_NODES, 32), f32),
    )(s1, dis, gcn1_b.reshape(1, 64), gcn2_W)

    s2 = _sc_aggregate(y2.reshape(NC * N_NODES, 32), srcs, dst_r)

    cls_wp = jnp.pad(cls_W, ((0, 0), (0, 118)))
    cls_bp = jnp.pad(cls_b, (0, 118)).reshape(1, 128)
    outp = pl.pallas_call(
        _head_body,
        grid=(B,),
        in_specs=[pl.BlockSpec((2, 2048, 32), lambda b: (0, b, 0)),
                  pl.BlockSpec((2048, 1), lambda b: (b, 0)),
                  pl.BlockSpec((1, 64), lambda b: (0, 0)),
                  pl.BlockSpec((64, 128), lambda b: (0, 0)),
                  pl.BlockSpec((1, 128), lambda b: (0, 0))],
        out_specs=pl.BlockSpec((1, 1, 128), lambda b: (b, 0, 0)),
        out_shape=jax.ShapeDtypeStruct((B, 1, 128), f32),
    )(s2, dis, gcn2_b.reshape(1, 64), cls_wp, cls_bp)

    return outp[:, 0, :10]
